# baseline (speedup 1.0000x reference)
import functools

import jax
import jax.numpy as jnp
from jax.experimental import pallas as pl
from jax.experimental.pallas import tpu as pltpu


def _prior_kernel(x_ref, mem_ref, out_ref, esums_ref, s0s_ref, *, b, n, m, d, bb):
    i = pl.program_id(0)
    scale = 1.0 / jnp.sqrt(jnp.float32(d))
    ones_row = jnp.ones((1, m), dtype=jnp.float32)
    sum_parts = []
    s0_parts = []
    for k in range(bb):
        x = x_ref[k]
        mem = mem_ref[k]

        mem_x0 = jnp.concatenate([mem, x[0:1, :]], axis=0)
        stx = jax.lax.dot_general(mem_x0, x, (((1,), (1,)), ((), ())),
                                  preferred_element_type=jnp.float32) * scale
        st = stx[0:m, :]
        s0_parts.append(stx[m:m + 1, :])

        smax = jnp.max(st, axis=0, keepdims=True)
        e = jnp.exp(st - smax)
        esum = jnp.sum(e, axis=0, keepdims=True)
        sum_parts.append(esum)

    esums_ref[pl.ds(i * bb, bb), :] = jnp.concatenate(sum_parts, axis=0)
    s0s_ref[pl.ds(i * bb, bb), :] = jnp.concatenate(s0_parts, axis=0)

    @pl.when(i == (b // bb) - 1)
    def _tail():
        esums = esums_ref[...]
        s0s = s0s_ref[...]
        xm_max = 1.0 / esums
        e0 = jnp.exp(s0s - jnp.max(s0s, axis=1, keepdims=True))
        xx0 = e0 / jnp.sum(e0, axis=1, keepdims=True)

        a = xx0 + xm_max
        lane = jax.lax.broadcasted_iota(jnp.int32, (b, n), 1)
        a = jnp.where(lane == 0, 0.0, a)
        amax = jnp.max(a, axis=1, keepdims=True)
        c = jnp.min(jnp.where(a == amax, lane, n), axis=1, keepdims=True)

        out = jnp.where(lane == 0, 0, c)
        out = jnp.where(lane == c, n - 1, out)
        out_ref[...] = out


def kernel(x, memory, src_mask, tgt_mask):
    b, n, d = x.shape
    m = memory.shape[1]
    bb = 8
    out = pl.pallas_call(
        functools.partial(_prior_kernel, b=b, n=n, m=m, d=d, bb=bb),
        grid=(b // bb,),
        in_specs=[
            pl.BlockSpec((bb, n, d), lambda i: (i, 0, 0)),
            pl.BlockSpec((bb, m, d), lambda i: (i, 0, 0)),
        ],
        out_specs=pl.BlockSpec((b, n), lambda i: (0, 0)),
        out_shape=jax.ShapeDtypeStruct((b, n), jnp.int32),
        scratch_shapes=[
            pltpu.VMEM((b, n), jnp.float32),
            pltpu.VMEM((b, n), jnp.float32),
        ],
    )(x, memory)
    return out


# speedup vs baseline: 1.0269x; 1.0269x over previous
import functools

import jax
import jax.numpy as jnp
from jax.experimental import pallas as pl
from jax.experimental.pallas import tpu as pltpu


def _batch_stats(x, mem, m, scale):
    mem_x0 = jnp.concatenate([mem, x[0:1, :]], axis=0)
    stx = jax.lax.dot_general(mem_x0, x, (((1,), (1,)), ((), ())),
                              preferred_element_type=jnp.float32) * scale
    st = stx[0:m, :]
    smax = jnp.max(st, axis=0, keepdims=True)
    esum = jnp.sum(jnp.exp(st - smax), axis=0, keepdims=True)
    return esum, stx[m:m + 1, :]


def _prior_kernel(xa_ref, xb_ref, ma_ref, mb_ref, out_ref, esums_ref, s0s_ref,
                  *, b, n, m, d, bb, steps):
    i = pl.program_id(0)
    scale = 1.0 / jnp.sqrt(jnp.float32(d))
    h = b // 2
    for part, (x_ref, mem_ref) in enumerate([(xa_ref, ma_ref), (xb_ref, mb_ref)]):
        sum_parts = []
        s0_parts = []
        for k in range(bb):
            esum, s0 = _batch_stats(x_ref[k], mem_ref[k], m, scale)
            sum_parts.append(esum)
            s0_parts.append(s0)
        base = part * h
        esums_ref[pl.ds(base + i * bb, bb), :] = jnp.concatenate(sum_parts, axis=0)
        s0s_ref[pl.ds(base + i * bb, bb), :] = jnp.concatenate(s0_parts, axis=0)

    @pl.when(i == steps - 1)
    def _tail():
        esums = esums_ref[...]
        s0s = s0s_ref[...]
        xm_max = 1.0 / esums
        e0 = jnp.exp(s0s - jnp.max(s0s, axis=1, keepdims=True))
        xx0 = e0 / jnp.sum(e0, axis=1, keepdims=True)

        a = xx0 + xm_max
        lane = jax.lax.broadcasted_iota(jnp.int32, (b, n), 1)
        a = jnp.where(lane == 0, 0.0, a)
        amax = jnp.max(a, axis=1, keepdims=True)
        c = jnp.min(jnp.where(a == amax, lane, n), axis=1, keepdims=True)

        out = jnp.where(lane == 0, 0, c)
        out = jnp.where(lane == c, n - 1, out)
        out_ref[...] = out


def kernel(x, memory, src_mask, tgt_mask):
    b, n, d = x.shape
    m = memory.shape[1]
    bb = 8
    steps = (b // 2) // bb

    def lo(i):
        return (i, 0, 0)

    def hi(i, _steps=steps):
        return (i + _steps, 0, 0)

    out = pl.pallas_call(
        functools.partial(_prior_kernel, b=b, n=n, m=m, d=d, bb=bb, steps=steps),
        grid=(steps,),
        in_specs=[
            pl.BlockSpec((bb, n, d), lo),
            pl.BlockSpec((bb, n, d), hi),
            pl.BlockSpec((bb, m, d), lo),
            pl.BlockSpec((bb, m, d), hi),
        ],
        out_specs=pl.BlockSpec((b, n), lambda i: (0, 0)),
        out_shape=jax.ShapeDtypeStruct((b, n), jnp.int32),
        scratch_shapes=[
            pltpu.VMEM((b, n), jnp.float32),
            pltpu.VMEM((b, n), jnp.float32),
        ],
    )(x, x, memory, memory)
    return out
